# baseline (speedup 1.0000x reference)
import functools

import jax
import jax.numpy as jnp
from jax import lax
from jax.experimental import pallas as pl
from jax.experimental.pallas import tpu as pltpu
from jax.experimental.pallas import tpu_sc as plsc

_LANES = 16
_SCALE = 8.0


def _emb_call(B, V, D, NW, CH, n_chunks):
    mesh = plsc.VectorSubcoreMesh(core_axis_name="c", subcore_axis_name="s")
    num_cores = mesh.num_cores

    @functools.partial(
        pl.kernel,
        out_type=jax.ShapeDtypeStruct((B, D), jnp.float32),
        mesh=mesh,
        scratch_types=[
            pltpu.VMEM((n_chunks, CH), jnp.int32),
            pltpu.VMEM((CH, D), jnp.float32),
            pltpu.SemaphoreType.DMA,
        ],
        compiler_params=pltpu.CompilerParams(use_tc_tiling_on_sc=False),
    )
    def emb_kernel(idx_hbm, table_hbm, out_hbm, idx_v, rows_v, sem):
        wid = lax.axis_index("s") * num_cores + lax.axis_index("c")
        pltpu.sync_copy(idx_hbm.at[wid], idx_v)

        @pl.loop(0, n_chunks)
        def _chunk(j):
            pltpu.async_copy(table_hbm.at[idx_v.at[j]], rows_v, sem).wait()

            @pl.loop(0, CH)
            def _row(i):
                for d in range(D // _LANES):
                    sl = pl.ds(d * _LANES, _LANES)
                    rows_v[i, sl] = rows_v[i, sl] * _SCALE

            base = (wid * n_chunks + j) * CH
            pltpu.sync_copy(rows_v, out_hbm.at[pl.ds(base, CH)])

    return emb_kernel


def kernel(inputs, table):
    B0, B1 = inputs.shape
    V, D = table.shape
    B = B0 * B1
    NW = 32
    CH = 128
    n_chunks = B // (NW * CH)

    idx = inputs.reshape(NW, n_chunks, CH).astype(jnp.int32)
    out = _emb_call(B, V, D, NW, CH, n_chunks)(idx, table)
    return out.reshape(B0, B1, D)
